# baseline (speedup 1.0000x reference)
import functools

import jax
import jax.numpy as jnp
from jax import lax
from jax.experimental import pallas as pl
from jax.experimental.pallas import tpu as pltpu
from jax.experimental.pallas import tpu_sc as plsc

_L = 16
_ROWS_PER_WORKER = 32
_NW = 32


def _sc_body(row_hbm, col_hbm, time_hbm, out_hbm,
             row_v, col_v, time_v, buf0, buf1, sem0, sem1):
    wid = lax.axis_index("s") * 2 + lax.axis_index("c")

    pltpu.sync_copy(row_hbm, row_v)
    pltpu.sync_copy(col_hbm, col_v)
    pltpu.sync_copy(time_hbm, time_v)

    start = jnp.minimum(wid * _ROWS_PER_WORKER, 1000 - _ROWS_PER_WORKER)

    bufs = (buf0, buf1)
    sems = (sem0, sem1)

    def fill(buf, r):
        w = r % 10
        h = (r // 10) % 10
        f = r // 100
        vecs = (
            [row_v[pl.ds(64 * w + _L * j, _L)] for j in range(4)]
            + [col_v[pl.ds(64 * h + _L * j, _L)] for j in range(4)]
            + [time_v[pl.ds(128 * f + _L * j, _L)] for j in range(8)]
        )

        def rep(b, carry):
            for j in range(16):
                buf[b, pl.ds(_L * j, _L)] = vecs[j]
            return carry

        lax.fori_loop(0, 64, rep, 0)

    for i in range(_ROWS_PER_WORKER):
        slot = i % 2
        r = start + i
        if i >= 2:
            pltpu.make_async_copy(bufs[slot], out_hbm.at[r - 2],
                                  sems[slot]).wait()
        fill(bufs[slot], r)
        pltpu.make_async_copy(bufs[slot], out_hbm.at[r], sems[slot]).start()

    for i in range(_ROWS_PER_WORKER - 2, _ROWS_PER_WORKER):
        slot = i % 2
        r = start + i
        pltpu.make_async_copy(bufs[slot], out_hbm.at[r], sems[slot]).wait()


def _make_sc_call():
    mesh = plsc.VectorSubcoreMesh(core_axis_name="c", subcore_axis_name="s")
    return functools.partial(
        pl.kernel,
        mesh=mesh,
        out_type=jax.ShapeDtypeStruct((1000, 64, 256), jnp.float32),
        scratch_types=[
            pltpu.VMEM((640,), jnp.float32),
            pltpu.VMEM((640,), jnp.float32),
            pltpu.VMEM((1280,), jnp.float32),
            pltpu.VMEM((64, 256), jnp.float32),
            pltpu.VMEM((64, 256), jnp.float32),
            pltpu.SemaphoreType.DMA,
            pltpu.SemaphoreType.DMA,
        ],
    )(_sc_body)


def kernel(x, row_embed, col_embed, time_embed):
    bs, frame_num, h, w = x.shape[:4]
    d4 = row_embed.shape[1]
    d2 = time_embed.shape[1]
    d = 2 * d4 + d2

    out = _make_sc_call()(row_embed.reshape(-1), col_embed.reshape(-1),
                          time_embed.reshape(-1))
    out = out.reshape(frame_num, h, w, bs, d)
    return jnp.transpose(out, (3, 0, 1, 2, 4))


# speedup vs baseline: 1.6250x; 1.6250x over previous
import jax
import jax.numpy as jnp
from jax import lax
from jax.experimental import pallas as pl
from jax.experimental.pallas import tpu as pltpu


def _pos_body(t_ref, o_ref):
    r, bs, d = o_ref.shape
    base = pl.program_id(0) * r
    rids = base + lax.broadcasted_iota(jnp.int32, (r, 32), 0)
    cids = lax.broadcasted_iota(jnp.int32, (r, 32), 1)
    sel = (cids == rids % 10)
    sel |= (cids == 10 + (rids // 10) % 10)
    sel |= (cids == 20 + rids // 100)
    s = sel.astype(jnp.float32)
    pos = jax.lax.dot_general(
        s, t_ref[...],
        dimension_numbers=(((1,), (0,)), ((), ())),
        preferred_element_type=jnp.float32,
        precision=jax.lax.Precision.HIGHEST,
    )
    o_ref[...] = jnp.broadcast_to(pos[:, None, :], (r, bs, d))


def kernel(x, row_embed, col_embed, time_embed):
    bs, frame_num, h, w = x.shape[:4]
    d4 = row_embed.shape[1]
    d2 = time_embed.shape[1]
    d = 2 * d4 + d2
    n = frame_num * h * w

    t = jnp.zeros((32, d), jnp.float32)
    t = t.at[0:10, 0:d4].set(row_embed)
    t = t.at[10:20, d4:2 * d4].set(col_embed)
    t = t.at[20:30, 2 * d4:d].set(time_embed)

    r = 40
    out = pl.pallas_call(
        _pos_body,
        grid=(n // r,),
        in_specs=[pl.BlockSpec((32, d), lambda i: (0, 0))],
        out_specs=pl.BlockSpec((r, bs, d), lambda i: (i, 0, 0)),
        out_shape=jax.ShapeDtypeStruct((n, bs, d), jnp.float32),
    )(t)
    out = out.reshape(frame_num, h, w, bs, d)
    return jnp.transpose(out, (3, 0, 1, 2, 4))


# speedup vs baseline: 2.1992x; 1.3534x over previous
import jax
import jax.numpy as jnp
from jax import lax
from jax.experimental import pallas as pl


def _pos_body(row_ref, col_ref, time_ref, o_ref):
    r, bs, d = o_ref.shape
    base = pl.program_id(0) * r
    rids = base + lax.broadcasted_iota(jnp.int32, (r, 16), 0)
    cids = lax.broadcasted_iota(jnp.int32, (r, 16), 1)

    def onehot_mm(idx, tbl):
        s = (cids == idx).astype(jnp.float32)
        return jax.lax.dot_general(
            s[:, :10], tbl,
            dimension_numbers=(((1,), (0,)), ((), ())),
            preferred_element_type=jnp.float32,
            precision=jax.lax.Precision.HIGHEST,
        )

    pos = jnp.concatenate(
        [
            onehot_mm(rids % 10, row_ref[...]),
            onehot_mm((rids // 10) % 10, col_ref[...]),
            onehot_mm(rids // 100, time_ref[...]),
        ],
        axis=-1,
    )
    o_ref[...] = jnp.broadcast_to(pos[:, None, :], (r, bs, d))


def kernel(x, row_embed, col_embed, time_embed):
    bs, frame_num, h, w = x.shape[:4]
    d4 = row_embed.shape[1]
    d2 = time_embed.shape[1]
    d = 2 * d4 + d2
    n = frame_num * h * w

    r = 100
    out = pl.pallas_call(
        _pos_body,
        grid=(n // r,),
        in_specs=[
            pl.BlockSpec((10, d4), lambda i: (0, 0)),
            pl.BlockSpec((10, d4), lambda i: (0, 0)),
            pl.BlockSpec((10, d2), lambda i: (0, 0)),
        ],
        out_specs=pl.BlockSpec((r, bs, d), lambda i: (i, 0, 0)),
        out_shape=jax.ShapeDtypeStruct((n, bs, d), jnp.float32),
    )(row_embed, col_embed, time_embed)
    out = out.reshape(frame_num, h, w, bs, d)
    return jnp.transpose(out, (3, 0, 1, 2, 4))
